# Initial kernel scaffold; baseline (speedup 1.0000x reference)
import jax
import jax.numpy as jnp
from jax.experimental import pallas as pl


def kernel(x, edge_index, topology_features, W_lin, b_lin, W_topo, b_topo, att_node, att_topology, bias):
    raise NotImplementedError("write your pallas kernel here")


# speedup vs baseline: 8.6872x; 8.6872x over previous
import functools

import jax
import jax.numpy as jnp
from jax import lax
from jax.experimental import pallas as pl
from jax.experimental.pallas import tpu as pltpu
from jax.experimental.pallas import tpu_sc as plsc

N = 10000
E = 320000
C = 128
NC, NS = 2, 16
CHUNK = 128
NCHUNK = E // CHUNK
NZCH = N // CHUNK
NTAIL = N - NZCH * CHUNK


def _proj_body(x_ref, t_ref, wl_ref, bl_ref, wt_ref, bt_ref, av_ref, tv_ref,
               y_ref, w_ref):
    xt = jnp.dot(x_ref[...], wl_ref[...],
                 preferred_element_type=jnp.float32) + bl_ref[...]
    tp = jnp.dot(t_ref[...], wt_ref[...],
                 preferred_element_type=jnp.float32) + bt_ref[...]
    a = (jnp.dot(xt, av_ref[...], preferred_element_type=jnp.float32)
         + jnp.dot(tp, tv_ref[...], preferred_element_type=jnp.float32))
    l = jnp.where(a >= 0.0, a, 0.2 * a)
    w = jnp.exp(l - jnp.max(l))
    y_ref[...] = xt * w
    w_ref[...] = w


_proj = pl.pallas_call(
    _proj_body,
    out_shape=[jax.ShapeDtypeStruct((N, C), jnp.float32),
               jax.ShapeDtypeStruct((N, 1), jnp.float32)],
)


def _scatter_body(row_hbm, col_hbm, y_hbm, out0, out1, cnt0, cnt1,
                  col_v, row_v, rows_v, ones_v, zc_v, acc_s, cnt_s, gsem):
    c = lax.axis_index("c")
    s = lax.axis_index("s")
    wid = s * NC + c

    def _zrow(r, _):
        for j in range(C // 16):
            rows_v[r, pl.ds(j * 16, 16)] = jnp.zeros((16,), jnp.float32)
        return 0
    lax.fori_loop(0, CHUNK, _zrow, 0)

    def _zc(i, _):
        zc_v[pl.ds(i * 16, 16)] = jnp.zeros((16,), jnp.float32)
        return 0
    lax.fori_loop(0, 64, _zc, 0)

    def _ones(i, _):
        ones_v[pl.ds(i * 16, 16)] = jnp.ones((16,), jnp.float32)
        return 0
    lax.fori_loop(0, CHUNK // 16, _ones, 0)

    def _zacc(i, _):
        off = pl.multiple_of((i * NS + s) * CHUNK, CHUNK)
        pltpu.sync_copy(rows_v, acc_s.at[pl.ds(off, CHUNK)])
        return 0
    lax.fori_loop(0, (NZCH - s + NS - 1) // NS, _zacc, 0)

    @pl.when(s == 0)
    def _():
        pltpu.sync_copy(rows_v.at[pl.ds(0, NTAIL)],
                        acc_s.at[pl.ds(NZCH * CHUNK, NTAIL)])

    @pl.when(s < N // 1000)
    def _():
        off = pl.multiple_of(s * 1000, 8)
        pltpu.sync_copy(zc_v.at[pl.ds(0, 1000)], cnt_s.at[pl.ds(off, 1000)])

    plsc.subcore_barrier()

    n_i = (NCHUNK - wid + NC * NS - 1) // (NC * NS)

    def _body(i, _):
        off = pl.multiple_of((i * NC * NS + wid) * CHUNK, CHUNK)
        pltpu.sync_copy(col_hbm.at[pl.ds(off, CHUNK)], col_v.at[0])
        pltpu.sync_copy(row_hbm.at[pl.ds(off, CHUNK)], row_v.at[0])
        pltpu.async_copy(y_hbm.at[col_v.at[0]], rows_v, gsem).wait()
        pltpu.sync_copy(rows_v, acc_s.at[row_v.at[0]], add=True)
        pltpu.sync_copy(ones_v, cnt_s.at[col_v.at[0]], add=True)
        return 0
    lax.fori_loop(0, n_i, _body, 0)

    plsc.subcore_barrier()

    def _wb(out_hbm, cnt_hbm):
        def _w(i, _):
            off = pl.multiple_of((i * NS + s) * CHUNK, CHUNK)
            pltpu.sync_copy(acc_s.at[pl.ds(off, CHUNK)],
                            out_hbm.at[pl.ds(off, CHUNK)])
            return 0
        lax.fori_loop(0, (NZCH - s + NS - 1) // NS, _w, 0)

        @pl.when(s == 0)
        def _():
            pltpu.sync_copy(acc_s.at[pl.ds(NZCH * CHUNK, NTAIL)],
                            out_hbm.at[pl.ds(NZCH * CHUNK, NTAIL)])

        @pl.when(s == 1)
        def _():
            pltpu.sync_copy(cnt_s, cnt_hbm)

    @pl.when(c == 0)
    def _():
        _wb(out0, cnt0)

    @pl.when(c == 1)
    def _():
        _wb(out1, cnt1)


@functools.lru_cache(maxsize=1)
def _get_scatter():
    mesh = plsc.VectorSubcoreMesh(core_axis_name="c", subcore_axis_name="s",
                                  num_cores=NC, num_subcores=NS)
    return pl.kernel(
        _scatter_body,
        out_type=[jax.ShapeDtypeStruct((N, C), jnp.float32),
                  jax.ShapeDtypeStruct((N, C), jnp.float32),
                  jax.ShapeDtypeStruct((N,), jnp.float32),
                  jax.ShapeDtypeStruct((N,), jnp.float32)],
        mesh=mesh,
        scratch_types=[
            pltpu.VMEM((2, CHUNK), jnp.int32),
            pltpu.VMEM((2, CHUNK), jnp.int32),
            pltpu.VMEM((CHUNK, C), jnp.float32),
            pltpu.VMEM((CHUNK,), jnp.float32),
            pltpu.VMEM((1024,), jnp.float32),
            pltpu.VMEM_SHARED((N, C), jnp.float32),
            pltpu.VMEM_SHARED((N,), jnp.float32),
            pltpu.SemaphoreType.DMA,
        ],
    )


def _combine_body(p0_ref, p1_ref, c0_ref, c1_ref, w_ref, b_ref, out_ref):
    z = jnp.dot(c0_ref[...] + c1_ref[...], w_ref[...],
                preferred_element_type=jnp.float32)
    out_ref[...] = (p0_ref[...] + p1_ref[...]) * (1.0 / z) + b_ref[...]


_combine = pl.pallas_call(
    _combine_body,
    out_shape=jax.ShapeDtypeStruct((N, C), jnp.float32),
)


def kernel(x, edge_index, topology_features, W_lin, b_lin, W_topo, b_topo,
           att_node, att_topology, bias):
    y, w = _proj(x[0], topology_features[0], W_lin, b_lin.reshape(1, -1),
                 W_topo, b_topo.reshape(1, -1), att_node.reshape(-1, 1),
                 att_topology.reshape(-1, 1))
    p0, p1, c0, c1 = _get_scatter()(edge_index[0], edge_index[1], y)
    out = _combine(p0, p1, c0.reshape(1, -1), c1.reshape(1, -1), w,
                   bias.reshape(1, -1))
    return out.reshape(1, N, -1), topology_features
